# Initial kernel scaffold; baseline (speedup 1.0000x reference)
import jax
import jax.numpy as jnp
from jax.experimental import pallas as pl


def kernel(x, pos, edge_index, W1, b1, W2, b2):
    raise NotImplementedError("write your pallas kernel here")


# speedup vs baseline: 5.5367x; 5.5367x over previous
import functools

import jax
import jax.numpy as jnp
from jax import lax
from jax.experimental import pallas as pl
from jax.experimental.pallas import tpu as pltpu
from jax.experimental.pallas import tpu_sc as plsc

NC = 2
NS = 16
NW = NC * NS
LANES = 16

CHUNK = 80


def _sc_segment_sum(x, src, dst, n, d, e):
    epw = e // NW
    n_chunks = epw // CHUNK
    n_pad = ((n + NS * 8 - 1) // (NS * 8)) * (NS * 8)
    rpt = n_pad // NS

    mesh = plsc.VectorSubcoreMesh(core_axis_name="c", subcore_axis_name="s")

    @functools.partial(
        pl.kernel,
        out_type=jax.ShapeDtypeStruct((NC, n_pad, d), jnp.float32),
        mesh=mesh,
        scratch_types=[
            pltpu.VMEM_SHARED((n_pad, d), jnp.float32),
            pltpu.VMEM((2, CHUNK), jnp.int32),
            pltpu.VMEM((2, CHUNK), jnp.int32),
            pltpu.VMEM((CHUNK, d), jnp.float32),
            pltpu.SemaphoreType.DMA,
        ],
    )
    def seg_sum(x_hbm, src_hbm, dst_hbm, out_hbm, acc, src_idx, dst_idx, rows, gsem):
        cid = lax.axis_index("c")
        sid = lax.axis_index("s")
        wid = sid * NC + cid

        zeros = jnp.zeros((LANES,), jnp.float32)

        def zero_row(i, carry):
            for j in range(d // LANES):
                rows[i, pl.ds(j * LANES, LANES)] = zeros
            return carry

        lax.fori_loop(0, CHUNK, zero_row, 0)
        done = 0
        while done < rpt:
            step = min(CHUNK, rpt - done)
            pltpu.sync_copy(
                rows.at[pl.ds(0, step)], acc.at[pl.ds(sid * rpt + done, step)]
            )
            done += step
        plsc.subcore_barrier()

        base_w = wid * epw

        def chunk_body(i, carry):
            base = base_w + i * CHUNK
            pltpu.sync_copy(src_hbm.at[pl.ds(base, CHUNK)], src_idx.at[0])
            pltpu.sync_copy(dst_hbm.at[pl.ds(base, CHUNK)], dst_idx.at[0])
            pltpu.async_copy(x_hbm.at[src_idx.at[0]], rows, gsem).wait()
            pltpu.sync_copy(rows, acc.at[dst_idx.at[0]], add=True)
            return carry

        lax.fori_loop(0, n_chunks, chunk_body, 0)
        plsc.subcore_barrier()

        pltpu.sync_copy(
            acc.at[pl.ds(sid * rpt, rpt)], out_hbm.at[cid, pl.ds(sid * rpt, rpt)]
        )

    return seg_sum(x, src, dst)


def _tc_mlp(x, partials, w1, b1, w2, b2, n, d):
    blk = 1000

    def mlp_body(x_ref, p_ref, w1_ref, b1_ref, w2_ref, b2_ref, o_ref):
        h = x_ref[...] + p_ref[0] + p_ref[1]
        h = jnp.dot(h, w1_ref[...], preferred_element_type=jnp.float32)
        h = jnp.maximum(h + b1_ref[...], 0.0)
        h = jnp.dot(h, w2_ref[...], preferred_element_type=jnp.float32)
        o_ref[...] = jnp.maximum(h + b2_ref[...], 0.0)

    return pl.pallas_call(
        mlp_body,
        grid=(n // blk,),
        in_specs=[
            pl.BlockSpec((blk, d), lambda i: (i, 0)),
            pl.BlockSpec((NC, blk, d), lambda i: (0, i, 0)),
            pl.BlockSpec((d, d), lambda i: (0, 0)),
            pl.BlockSpec((1, d), lambda i: (0, 0)),
            pl.BlockSpec((d, d), lambda i: (0, 0)),
            pl.BlockSpec((1, d), lambda i: (0, 0)),
        ],
        out_specs=pl.BlockSpec((blk, d), lambda i: (i, 0)),
        out_shape=jax.ShapeDtypeStruct((n, d), jnp.float32),
    )(x, partials, w1, b1.reshape(1, d), w2, b2.reshape(1, d))


def kernel(x, pos, edge_index, W1, b1, W2, b2):
    n, d = x.shape
    e = edge_index.shape[1]
    src = edge_index[0]
    dst = edge_index[1]
    partials = _sc_segment_sum(x, src, dst, n, d, e)
    return _tc_mlp(x, partials, W1, b1, W2, b2, n, d)


# speedup vs baseline: 12.1121x; 2.1876x over previous
import functools

import jax
import jax.numpy as jnp
from jax import lax
from jax.experimental import pallas as pl
from jax.experimental.pallas import tpu as pltpu
from jax.experimental.pallas import tpu_sc as plsc

NC = 2
NS = 16
NW = NC * NS
LANES = 16

CHUNK = 80


def _sc_segment_sum(x, src, dst, n, d, e):
    epw = e // NW
    n_chunks = epw // CHUNK
    n_pad = ((n + NS * 8 - 1) // (NS * 8)) * (NS * 8)
    rpt = n_pad // NS

    mesh = plsc.VectorSubcoreMesh(core_axis_name="c", subcore_axis_name="s")

    @functools.partial(
        pl.kernel,
        out_type=jax.ShapeDtypeStruct((NC, n_pad, d), jnp.float32),
        mesh=mesh,
        scratch_types=[
            pltpu.VMEM_SHARED((n_pad, d), jnp.float32),
            pltpu.VMEM((epw,), jnp.int32),
            pltpu.VMEM((n_chunks, CHUNK), jnp.int32),
            pltpu.VMEM((2, CHUNK, d), jnp.float32),
            pltpu.SemaphoreType.DMA((2,)),
            pltpu.SemaphoreType.DMA,
        ],
    )
    def seg_sum(x_hbm, src_hbm, dst_hbm, out_hbm, acc, src_idx, dst_idx, rows, gsem, isem):
        cid = lax.axis_index("c")
        sid = lax.axis_index("s")
        wid = sid * NC + cid

        src_cp = pltpu.async_copy(src_hbm.at[wid], src_idx, isem)
        dst_cp = pltpu.async_copy(dst_hbm.at[wid], dst_idx, isem)

        zeros = jnp.zeros((LANES,), jnp.float32)

        def zero_row(i, carry):
            for j in range(d // LANES):
                rows[0, i, pl.ds(j * LANES, LANES)] = zeros
            return carry

        lax.fori_loop(0, CHUNK, zero_row, 0)
        done = 0
        while done < rpt:
            step = min(CHUNK, rpt - done)
            pltpu.sync_copy(
                rows.at[0, pl.ds(0, step)], acc.at[pl.ds(sid * rpt + done, step)]
            )
            done += step
        src_cp.wait()
        dst_cp.wait()
        plsc.subcore_barrier()

        def src_slice(i):
            return src_idx.at[pl.ds(pl.multiple_of(i * CHUNK, 8), CHUNK)]

        pltpu.async_copy(x_hbm.at[src_slice(0)], rows.at[0], gsem.at[0])

        def chunk_body(i, carry):
            b = lax.rem(i, 2)
            nb = 1 - b

            @pl.when(i + 1 < n_chunks)
            def _():
                pltpu.async_copy(x_hbm.at[src_slice(i + 1)], rows.at[nb], gsem.at[nb])

            pltpu.make_async_copy(x_hbm.at[src_slice(i)], rows.at[b], gsem.at[b]).wait()
            pltpu.sync_copy(rows.at[b], acc.at[dst_idx.at[i]], add=True)
            return carry

        lax.fori_loop(0, n_chunks, chunk_body, 0)
        plsc.subcore_barrier()

        pltpu.sync_copy(
            acc.at[pl.ds(sid * rpt, rpt)], out_hbm.at[cid, pl.ds(sid * rpt, rpt)]
        )

    return seg_sum(
        x,
        src.reshape(NW, epw),
        dst.reshape(NW, n_chunks, CHUNK),
    )


def _tc_mlp(x, partials, w1, b1, w2, b2, n, d):
    blk = 1000

    def mlp_body(x_ref, p_ref, w1_ref, b1_ref, w2_ref, b2_ref, o_ref):
        h = x_ref[...] + p_ref[0] + p_ref[1]
        h = jnp.dot(h, w1_ref[...], preferred_element_type=jnp.float32)
        h = jnp.maximum(h + b1_ref[...], 0.0)
        h = jnp.dot(h, w2_ref[...], preferred_element_type=jnp.float32)
        o_ref[...] = jnp.maximum(h + b2_ref[...], 0.0)

    return pl.pallas_call(
        mlp_body,
        grid=(n // blk,),
        in_specs=[
            pl.BlockSpec((blk, d), lambda i: (i, 0)),
            pl.BlockSpec((NC, blk, d), lambda i: (0, i, 0)),
            pl.BlockSpec((d, d), lambda i: (0, 0)),
            pl.BlockSpec((1, d), lambda i: (0, 0)),
            pl.BlockSpec((d, d), lambda i: (0, 0)),
            pl.BlockSpec((1, d), lambda i: (0, 0)),
        ],
        out_specs=pl.BlockSpec((blk, d), lambda i: (i, 0)),
        out_shape=jax.ShapeDtypeStruct((n, d), jnp.float32),
    )(x, partials, w1, b1.reshape(1, d), w2, b2.reshape(1, d))


def kernel(x, pos, edge_index, W1, b1, W2, b2):
    n, d = x.shape
    e = edge_index.shape[1]
    src = edge_index[0]
    dst = edge_index[1]
    partials = _sc_segment_sum(x, src, dst, n, d, e)
    return _tc_mlp(x, partials, W1, b1, W2, b2, n, d)


# speedup vs baseline: 13.9155x; 1.1489x over previous
import functools

import jax
import jax.numpy as jnp
from jax import lax
from jax.experimental import pallas as pl
from jax.experimental.pallas import tpu as pltpu
from jax.experimental.pallas import tpu_sc as plsc

NC = 2
NS = 16
NW = NC * NS
LANES = 16

CHUNK = 80


def _sc_segment_sum(x, src, dst, n, d, e):
    epw = e // NW
    n_chunks = epw // CHUNK
    n_pad = ((n + NS * 8 - 1) // (NS * 8)) * (NS * 8)
    rpt = n_pad // NS

    mesh = plsc.VectorSubcoreMesh(core_axis_name="c", subcore_axis_name="s")

    @functools.partial(
        pl.kernel,
        out_type=jax.ShapeDtypeStruct((NC, n_pad, d), jnp.float32),
        mesh=mesh,
        scratch_types=[
            pltpu.VMEM_SHARED((n_pad, d), jnp.float32),
            pltpu.VMEM((epw,), jnp.int32),
            pltpu.VMEM((3, CHUNK), jnp.int32),
            pltpu.VMEM((3, CHUNK, d), jnp.float32),
            pltpu.SemaphoreType.DMA((3,)),
            pltpu.SemaphoreType.DMA((3,)),
            pltpu.SemaphoreType.DMA((3,)),
            pltpu.SemaphoreType.DMA,
        ],
    )
    def seg_sum(x_hbm, src_hbm, dst_hbm, out_hbm, acc, src_idx, dstg, rows,
                gsem, ssem, dsem, isem):
        cid = lax.axis_index("c")
        sid = lax.axis_index("s")
        wid = sid * NC + cid

        base_w = pl.multiple_of(wid * epw, 8)
        src_cp = pltpu.async_copy(src_hbm.at[pl.ds(base_w, epw)], src_idx, isem)

        zeros = jnp.zeros((LANES,), jnp.float32)

        def zero_row(i, carry):
            for j in range(d // LANES):
                rows[0, i, pl.ds(j * LANES, LANES)] = zeros
            return carry

        lax.fori_loop(0, CHUNK, zero_row, 0)
        done = 0
        while done < rpt:
            step = min(CHUNK, rpt - done)
            pltpu.sync_copy(
                rows.at[0, pl.ds(0, step)], acc.at[pl.ds(sid * rpt + done, step)]
            )
            done += step
        src_cp.wait()
        plsc.subcore_barrier()

        def src_slice(i):
            return src_idx.at[pl.ds(pl.multiple_of(i * CHUNK, 8), CHUNK)]

        def dst_slice(i):
            return dst_hbm.at[pl.ds(pl.multiple_of(base_w + i * CHUNK, 8), CHUNK)]

        def issue_loads(j, jb):
            pltpu.async_copy(dst_slice(j), dstg.at[jb], dsem.at[jb])
            pltpu.async_copy(x_hbm.at[src_slice(j)], rows.at[jb], gsem.at[jb])

        for k in range(min(2, n_chunks)):
            issue_loads(k, k)

        def chunk_body(i, carry):
            b = lax.rem(i, 3)
            pltpu.make_async_copy(x_hbm.at[src_slice(i)], rows.at[b], gsem.at[b]).wait()
            pltpu.make_async_copy(dst_slice(i), dstg.at[b], dsem.at[b]).wait()
            pltpu.async_copy(rows.at[b], acc.at[dstg.at[b]], ssem.at[b], add=True)

            @pl.when(i + 2 < n_chunks)
            def _():
                pb = lax.rem(i + 2, 3)

                @pl.when(i >= 1)
                def _():
                    pltpu.make_async_copy(
                        rows.at[pb], acc.at[dstg.at[pb]], ssem.at[pb]
                    ).wait()

                issue_loads(i + 2, pb)

            return carry

        lax.fori_loop(0, n_chunks, chunk_body, 0)
        for k in range(max(0, n_chunks - 3), n_chunks):
            b = k % 3
            pltpu.make_async_copy(rows.at[b], acc.at[dstg.at[b]], ssem.at[b]).wait()
        plsc.subcore_barrier()

        pltpu.sync_copy(
            acc.at[pl.ds(sid * rpt, rpt)], out_hbm.at[cid, pl.ds(sid * rpt, rpt)]
        )

    return seg_sum(
        x,
        src,
        dst,
    )


def _tc_mlp(x, partials, w1, b1, w2, b2, n, d):
    blk = 1000

    def mlp_body(x_ref, p_ref, w1_ref, b1_ref, w2_ref, b2_ref, o_ref):
        h = x_ref[...] + p_ref[0] + p_ref[1]
        h = jnp.dot(h, w1_ref[...], preferred_element_type=jnp.float32)
        h = jnp.maximum(h + b1_ref[...], 0.0)
        h = jnp.dot(h, w2_ref[...], preferred_element_type=jnp.float32)
        o_ref[...] = jnp.maximum(h + b2_ref[...], 0.0)

    return pl.pallas_call(
        mlp_body,
        grid=(n // blk,),
        in_specs=[
            pl.BlockSpec((blk, d), lambda i: (i, 0)),
            pl.BlockSpec((NC, blk, d), lambda i: (0, i, 0)),
            pl.BlockSpec((d, d), lambda i: (0, 0)),
            pl.BlockSpec((1, d), lambda i: (0, 0)),
            pl.BlockSpec((d, d), lambda i: (0, 0)),
            pl.BlockSpec((1, d), lambda i: (0, 0)),
        ],
        out_specs=pl.BlockSpec((blk, d), lambda i: (i, 0)),
        out_shape=jax.ShapeDtypeStruct((n, d), jnp.float32),
    )(x, partials, w1, b1.reshape(1, d), w2, b2.reshape(1, d))


def kernel(x, pos, edge_index, W1, b1, W2, b2):
    n, d = x.shape
    e = edge_index.shape[1]
    src = edge_index[0]
    dst = edge_index[1]
    partials = _sc_segment_sum(x, src, dst, n, d, e)
    return _tc_mlp(x, partials, W1, b1, W2, b2, n, d)
